# baseline (speedup 1.0000x reference)
import functools

import jax
import jax.numpy as jnp
from jax import lax
from jax.experimental import pallas as pl
from jax.experimental.pallas import tpu as pltpu
from jax.experimental.pallas import tpu_sc as plsc

B, P, D = 64, 1024, 768
NC, NS = 2, 16
LANES = 16
PCHUNK = P // NS
BHALF = B // NC
SUB = 32
NSTEP = PCHUNK // SUB
CH = SUB * D
POSLEN = PCHUNK * D


def _sc_add(x_flat, pos_flat):
    mesh = plsc.VectorSubcoreMesh(core_axis_name="c", subcore_axis_name="s")

    @functools.partial(
        pl.kernel,
        out_type=jax.ShapeDtypeStruct((B * P * D,), jnp.float32),
        mesh=mesh,
        scratch_types=[
            pltpu.VMEM((POSLEN,), jnp.float32),
            pltpu.VMEM((CH,), jnp.float32),
        ],
    )
    def body(x_hbm, pos_hbm, out_hbm, pos_v, buf):
        c = lax.axis_index("c")
        s = lax.axis_index("s")
        p0 = s * PCHUNK
        b0 = c * BHALF
        pltpu.sync_copy(pos_hbm.at[pl.ds(p0 * D, POSLEN)], pos_v)

        def per_batch(b, _):
            row = (b0 + b) * P + p0
            for h in range(NSTEP):
                off = (row + h * SUB) * D
                poff = h * CH
                pltpu.sync_copy(x_hbm.at[pl.ds(off, CH)], buf)

                @plsc.parallel_loop(0, CH // LANES, 1, unroll=8)
                def _(i):
                    sl = pl.ds(i * LANES, LANES)
                    buf[sl] = buf[sl] + pos_v[pl.ds(poff + i * LANES, LANES)]

                pltpu.sync_copy(buf, out_hbm.at[pl.ds(off, CH)])
            return 0

        lax.fori_loop(0, BHALF, per_batch, 0)

    return body(x_flat, pos_flat)


def kernel(patch_embeddings, pos_table):
    out = _sc_add(patch_embeddings.reshape(-1), pos_table.reshape(-1))
    return out.reshape(B, P, D)


# speedup vs baseline: 1.2742x; 1.2742x over previous
import functools

import jax
import jax.numpy as jnp
from jax import lax
from jax.experimental import pallas as pl
from jax.experimental.pallas import tpu as pltpu
from jax.experimental.pallas import tpu_sc as plsc

B, P, D = 64, 1024, 768
NC, NS = 2, 16
LANES = 16
PCHUNK = P // NS
BHALF = B // NC
NBUF = 4
SUB = PCHUNK // NBUF
CH = SUB * D
POSLEN = PCHUNK * D


def _sc_add(x_flat, pos_flat):
    mesh = plsc.VectorSubcoreMesh(core_axis_name="c", subcore_axis_name="s")

    @functools.partial(
        pl.kernel,
        out_type=jax.ShapeDtypeStruct((B * P * D,), jnp.float32),
        mesh=mesh,
        scratch_types=[
            pltpu.VMEM((POSLEN,), jnp.float32),
            [pltpu.VMEM((CH,), jnp.float32)] * NBUF,
            [pltpu.SemaphoreType.DMA] * NBUF,
            [pltpu.SemaphoreType.DMA] * NBUF,
        ],
    )
    def body(x_hbm, pos_hbm, out_hbm, pos_v, bufs, gsem, ssem):
        c = lax.axis_index("c")
        s = lax.axis_index("s")
        p0 = s * PCHUNK
        b0 = c * BHALF
        pltpu.sync_copy(pos_hbm.at[pl.ds(p0 * D, POSLEN)], pos_v)

        def off(b, k):
            return ((b0 + b) * P + p0 + k * SUB) * D

        for k in range(NBUF - 1):
            pltpu.async_copy(x_hbm.at[pl.ds(off(0, k), CH)], bufs[k], gsem[k])

        def per_batch(u, _):
            for k in range(NBUF):
                pltpu.make_async_copy(
                    x_hbm.at[pl.ds(off(u, k), CH)], bufs[k], gsem[k]
                ).wait()

                buf = bufs[k]
                poff = k * CH

                @plsc.parallel_loop(0, CH // LANES, 1, unroll=8)
                def _(i):
                    sl = pl.ds(i * LANES, LANES)
                    buf[sl] = buf[sl] + pos_v[pl.ds(poff + i * LANES, LANES)]

                pltpu.async_copy(buf, out_hbm.at[pl.ds(off(u, k), CH)], ssem[k])

                kprev = (k - 1) % NBUF
                knext = (k + NBUF - 1) % NBUF
                bn, kn = (u, k + NBUF - 1) if k == 0 else (u + 1, k - 1)

                if k == 0:
                    @pl.when(u > 0)
                    def _():
                        pltpu.make_async_copy(
                            bufs[kprev], out_hbm.at[pl.ds(0, CH)], ssem[kprev]
                        ).wait()
                    pltpu.async_copy(
                        x_hbm.at[pl.ds(off(bn, kn), CH)], bufs[knext], gsem[knext]
                    )
                else:
                    pltpu.make_async_copy(
                        bufs[kprev], out_hbm.at[pl.ds(0, CH)], ssem[kprev]
                    ).wait()

                    @pl.when(u < BHALF - 1)
                    def _():
                        pltpu.async_copy(
                            x_hbm.at[pl.ds(off(bn, kn), CH)], bufs[knext], gsem[knext]
                        )
            return 0

        lax.fori_loop(0, BHALF, per_batch, 0)
        pltpu.make_async_copy(
            bufs[NBUF - 1], out_hbm.at[pl.ds(0, CH)], ssem[NBUF - 1]
        ).wait()

    return body(x_flat, pos_flat)


def kernel(patch_embeddings, pos_table):
    out = _sc_add(patch_embeddings.reshape(-1), pos_table.reshape(-1))
    return out.reshape(B, P, D)


# speedup vs baseline: 4.4024x; 3.4551x over previous
import functools

import jax
import jax.numpy as jnp
from jax import lax
from jax.experimental import pallas as pl
from jax.experimental.pallas import tpu as pltpu
from jax.experimental.pallas import tpu_sc as plsc

B, P, D = 64, 1024, 768
NC, NS = 2, 16
LANES = 16
PCHUNK = P // NS
BHALF = B // NC
NBUF = 4
SUB = PCHUNK // NBUF
POSLEN = PCHUNK * D


def kernel(patch_embeddings, pos_table):
    mesh = plsc.VectorSubcoreMesh(core_axis_name="c", subcore_axis_name="s")

    @functools.partial(
        pl.kernel,
        out_type=jax.ShapeDtypeStruct((B, P, D), jnp.float32),
        mesh=mesh,
        compiler_params=pltpu.CompilerParams(use_tc_tiling_on_sc=True),
        scratch_types=[
            pltpu.VMEM((PCHUNK, D), jnp.float32),
            [pltpu.VMEM((SUB, D), jnp.float32)] * NBUF,
            [pltpu.SemaphoreType.DMA] * NBUF,
            [pltpu.SemaphoreType.DMA] * NBUF,
        ],
    )
    def body(x_hbm, pos_hbm, out_hbm, pos_v, bufs, gsem, ssem):
        c = lax.axis_index("c")
        s = lax.axis_index("s")
        p0 = s * PCHUNK
        b0 = c * BHALF
        pltpu.sync_copy(pos_hbm.at[pl.ds(p0, PCHUNK)], pos_v)

        for k in range(NBUF - 1):
            pltpu.async_copy(
                x_hbm.at[b0, pl.ds(p0 + k * SUB, SUB)], bufs[k], gsem[k]
            )

        def per_batch(u, _):
            b = b0 + u
            for k in range(NBUF):
                pltpu.make_async_copy(
                    x_hbm.at[b, pl.ds(p0 + k * SUB, SUB)], bufs[k], gsem[k]
                ).wait()

                buf = bufs[k]

                @plsc.parallel_loop(0, SUB, 1)
                def _(r):
                    pr = k * SUB + r

                    @plsc.parallel_loop(0, D // LANES, 1, unroll=8)
                    def _(i):
                        sl = pl.ds(i * LANES, LANES)
                        buf[r, sl] = buf[r, sl] + pos_v[pr, sl]

                pltpu.async_copy(
                    buf, out_hbm.at[b, pl.ds(p0 + k * SUB, SUB)], ssem[k]
                )

                kprev = (k - 1) % NBUF
                bn, kn = (b, k + NBUF - 1) if k == 0 else (b + 1, k - 1)

                if k == 0:
                    @pl.when(u > 0)
                    def _():
                        pltpu.make_async_copy(
                            bufs[kprev],
                            out_hbm.at[b0, pl.ds(p0, SUB)],
                            ssem[kprev],
                        ).wait()
                    pltpu.async_copy(
                        x_hbm.at[bn, pl.ds(p0 + kn * SUB, SUB)],
                        bufs[kprev],
                        gsem[kprev],
                    )
                else:
                    pltpu.make_async_copy(
                        bufs[kprev],
                        out_hbm.at[b0, pl.ds(p0, SUB)],
                        ssem[kprev],
                    ).wait()

                    @pl.when(u < BHALF - 1)
                    def _():
                        pltpu.async_copy(
                            x_hbm.at[bn, pl.ds(p0 + kn * SUB, SUB)],
                            bufs[kprev],
                            gsem[kprev],
                        )
            return 0

        lax.fori_loop(0, BHALF, per_batch, 0)
        pltpu.make_async_copy(
            bufs[NBUF - 1], out_hbm.at[b0, pl.ds(p0, SUB)], ssem[NBUF - 1]
        ).wait()

    return body(patch_embeddings, pos_table)


# speedup vs baseline: 4.4126x; 1.0023x over previous
import functools

import jax
import jax.numpy as jnp
from jax import lax
from jax.experimental import pallas as pl
from jax.experimental.pallas import tpu as pltpu
from jax.experimental.pallas import tpu_sc as plsc

B, P, D = 64, 1024, 768
NC, NS = 2, 16
NW = NC * NS
LANES = 16
PW = P // NW
SUB = 16
NB = 2
NPAIR = 3
G = (B // NB) * (PW // SUB)


def kernel(patch_embeddings, pos_table):
    mesh = plsc.VectorSubcoreMesh(core_axis_name="c", subcore_axis_name="s")

    @functools.partial(
        pl.kernel,
        out_type=jax.ShapeDtypeStruct((B, P, D), jnp.float32),
        mesh=mesh,
        compiler_params=pltpu.CompilerParams(use_tc_tiling_on_sc=True),
        scratch_types=[
            pltpu.VMEM((PW, D), jnp.float32),
            [pltpu.VMEM((SUB, D), jnp.float32)] * (NB * NPAIR),
            [pltpu.SemaphoreType.DMA] * (NB * NPAIR),
            [pltpu.SemaphoreType.DMA] * (NB * NPAIR),
        ],
    )
    def body(x_hbm, pos_hbm, out_hbm, pos_v, bufs, gsem, ssem):
        c = lax.axis_index("c")
        s = lax.axis_index("s")
        w = s * NC + c
        p0 = w * PW
        pltpu.sync_copy(pos_hbm.at[pl.ds(p0, PW)], pos_v)

        def start_gather(g, e):
            h = g % 2
            b = (g // 2) * NB
            rows = pl.ds(p0 + h * SUB, SUB)
            for j in range(NB):
                pltpu.async_copy(
                    x_hbm.at[b + j, rows], bufs[NB * e + j], gsem[NB * e + j]
                )

        def run_group(g, e):
            h = g % 2
            b = (g // 2) * NB
            rows = pl.ds(p0 + h * SUB, SUB)
            for j in range(NB):
                pltpu.make_async_copy(
                    x_hbm.at[b + j, rows], bufs[NB * e + j], gsem[NB * e + j]
                ).wait()

            bA = bufs[NB * e]
            bB = bufs[NB * e + 1]
            prow = h * SUB

            @plsc.parallel_loop(0, SUB, 1)
            def _(r):
                pr = prow + r

                @plsc.parallel_loop(0, D // LANES, 1, unroll=4)
                def _(i):
                    sl = pl.ds(i * LANES, LANES)
                    pv = pos_v[pr, sl]
                    bA[r, sl] = bA[r, sl] + pv
                    bB[r, sl] = bB[r, sl] + pv

            for j in range(NB):
                pltpu.async_copy(
                    bufs[NB * e + j], out_hbm.at[b + j, rows], ssem[NB * e + j]
                )

        def wait_scatter_pair(e2):
            for j in range(NB):
                pltpu.make_async_copy(
                    bufs[NB * e2 + j],
                    out_hbm.at[0, pl.ds(0, SUB)],
                    ssem[NB * e2 + j],
                ).wait()

        start_gather(0, 0)
        start_gather(1, 1)

        def per_iter(u, _):
            for e in range(NPAIR):
                g = NPAIR * u + e
                e2 = (e + 2) % NPAIR
                run_group(g, e)
                if e == 0:
                    @pl.when(u > 0)
                    def _():
                        wait_scatter_pair(e2)
                    start_gather(g + 2, e2)
                else:
                    wait_scatter_pair(e2)

                    @pl.when(g + 2 < G)
                    def _():
                        start_gather(g + 2, e2)
            return 0

        lax.fori_loop(0, (G - 1) // NPAIR, per_iter, 0)

        run_group(G - 1, 0)
        wait_scatter_pair((0 + 2) % NPAIR)
        wait_scatter_pair(0)

    return body(patch_embeddings, pos_table)
